# baseline (speedup 1.0000x reference)
import jax
import jax.numpy as jnp
from jax.experimental import pallas as pl
from jax.experimental.pallas import tpu as pltpu

_CT_COLS = 200
_CT_ROWS = 8
_IND_COLS = 250
_IND_ROWS = 8


def _body(s_ref, ct_ref, ind_ref, t_ref, saved_ref, new_ref,
          ct_out, ind_out, saved_out):
    del saved_ref
    idx = s_ref[0]

    off = idx % (_IND_ROWS * _IND_COLS)
    r = off // _IND_COLS
    c = off - r * _IND_COLS
    row_i = jax.lax.broadcasted_iota(jnp.int32, (_IND_ROWS, _IND_COLS), 0)
    col_i = jax.lax.broadcasted_iota(jnp.int32, (_IND_ROWS, _IND_COLS), 1)
    hit = (row_i == r) & (col_i == c)
    ind_blk = ind_ref[...]
    cursor = jnp.sum(jnp.where(hit, ind_blk, 0))
    ind_out[...] = ind_blk + hit.astype(ind_blk.dtype)

    rr = idx % _CT_ROWS
    row_c = jax.lax.broadcasted_iota(jnp.int32, (_CT_ROWS, _CT_COLS), 0)
    col_c = jax.lax.broadcasted_iota(jnp.int32, (_CT_ROWS, _CT_COLS), 1)
    ct_out[...] = jnp.where((row_c == rr) & (col_c == cursor),
                            t_ref[0], ct_ref[...])

    saved_out[...] = new_ref[...]


def kernel(clicktimes, indices, idx, t, saved, new, save_index):
    idx32 = jnp.asarray(idx, jnp.int32)
    si32 = jnp.asarray(save_index, jnp.int32)
    s = jnp.stack([idx32, si32])
    t_arr = jnp.asarray(t, jnp.float32).reshape(1)
    ind2d = indices.reshape(-1, _IND_COLS)

    grid_spec = pltpu.PrefetchScalarGridSpec(
        num_scalar_prefetch=1,
        grid=(1,),
        in_specs=[
            pl.BlockSpec((_CT_ROWS, _CT_COLS),
                         lambda i, s: (s[0] // _CT_ROWS, 0)),
            pl.BlockSpec((_IND_ROWS, _IND_COLS),
                         lambda i, s: (s[0] // (_IND_ROWS * _IND_COLS), 0)),
            pl.BlockSpec(memory_space=pltpu.SMEM),
            pl.BlockSpec(memory_space=pltpu.HBM),
            pl.BlockSpec((1,) + new.shape[1:], lambda i, s: (s[1], 0, 0)),
        ],
        out_specs=[
            pl.BlockSpec((_CT_ROWS, _CT_COLS),
                         lambda i, s: (s[0] // _CT_ROWS, 0)),
            pl.BlockSpec((_IND_ROWS, _IND_COLS),
                         lambda i, s: (s[0] // (_IND_ROWS * _IND_COLS), 0)),
            pl.BlockSpec((1,) + new.shape[1:], lambda i, s: (s[1], 0, 0)),
        ],
    )
    ct_out, ind2d_out, saved_out = pl.pallas_call(
        _body,
        grid_spec=grid_spec,
        out_shape=[
            jax.ShapeDtypeStruct(clicktimes.shape, clicktimes.dtype),
            jax.ShapeDtypeStruct(ind2d.shape, ind2d.dtype),
            jax.ShapeDtypeStruct(saved.shape, saved.dtype),
        ],
        input_output_aliases={1: 0, 2: 1, 4: 2},
    )(s, clicktimes, ind2d, t_arr, saved, new)

    return (ct_out, ind2d_out.reshape(indices.shape), saved_out,
            save_index + 1)
